# baseline (speedup 1.0000x reference)
import jax
import jax.numpy as jnp
from jax.experimental import pallas as pl

N = 25000
NPAD = 26000
E = 150000
H = 128
R = 1000
NB = NPAD // R
NBO = N // R
DUMMY_DST = N


def _leaky(x):
    return jnp.where(x >= 0, x, 0.01 * x)


def _prep_body(xm_ref, w_ref, b_ref, out_ref):
    out_ref[...] = jnp.dot(xm_ref[...], w_ref[...],
                           preferred_element_type=jnp.float32) + b_ref[...]


def _prep_mtrans(x_m_pad, Wp, bp):
    return pl.pallas_call(
        _prep_body,
        grid=(NB,),
        in_specs=[
            pl.BlockSpec((R, 8), lambda i: (i, 0)),
            pl.BlockSpec((8, 16), lambda i: (0, 0)),
            pl.BlockSpec((1, 16), lambda i: (0, 0)),
        ],
        out_specs=pl.BlockSpec((R, 16), lambda i: (i, 0)),
        out_shape=jax.ShapeDtypeStruct((NPAD, 16), jnp.float32),
    )(x_m_pad, Wp, bp)


def _mlp_pair_body(a1_ref, a3_ref,
                   w1a_ref, b1a_ref, w1b_ref, b1b_ref,
                   w3a_ref, b3a_ref, w3b_ref, b3b_ref,
                   out_ref):
    h1 = jnp.maximum(jnp.dot(a1_ref[...], w1a_ref[...],
                             preferred_element_type=jnp.float32) + b1a_ref[...], 0.0)
    t1 = jnp.dot(h1, w1b_ref[...], preferred_element_type=jnp.float32) + b1b_ref[...]
    h3 = jnp.maximum(jnp.dot(a3_ref[...], w3a_ref[...],
                             preferred_element_type=jnp.float32) + b3a_ref[...], 0.0)
    t3 = jnp.dot(h3, w3b_ref[...], preferred_element_type=jnp.float32) + b3b_ref[...]
    out_ref[...] = _leaky(t1 + t3)


def _layer0_half(agg0_flat, t1, t3, w1a, b1a, w1b, b1b, w3a, b3a, w3b, b3b):
    return pl.pallas_call(
        _mlp_pair_body,
        grid=(NB,),
        in_specs=[
            pl.BlockSpec((R, 16), lambda i, t1=t1: (i + t1 * NB, 0)),
            pl.BlockSpec((R, 16), lambda i, t3=t3: (i + t3 * NB, 0)),
            pl.BlockSpec((16, H), lambda i: (0, 0)),
            pl.BlockSpec((1, H), lambda i: (0, 0)),
            pl.BlockSpec((H, H), lambda i: (0, 0)),
            pl.BlockSpec((1, H), lambda i: (0, 0)),
            pl.BlockSpec((16, H), lambda i: (0, 0)),
            pl.BlockSpec((1, H), lambda i: (0, 0)),
            pl.BlockSpec((H, H), lambda i: (0, 0)),
            pl.BlockSpec((1, H), lambda i: (0, 0)),
        ],
        out_specs=pl.BlockSpec((R, H), lambda i: (i, 0)),
        out_shape=jax.ShapeDtypeStruct((NPAD, H), jnp.float32),
    )(agg0_flat, agg0_flat, w1a, b1a, w1b, b1b, w3a, b3a, w3b, b3b)


def _mlp_pair_fc_body(a1lo_ref, a1hi_ref, a3lo_ref, a3hi_ref,
                      w1a_ref, b1a_ref, w1b_ref, b1b_ref,
                      w3a_ref, b3a_ref, w3b_ref, b3b_ref,
                      wfc_ref, bfc_ref, out_ref):
    u1 = (jnp.dot(a1lo_ref[...], w1a_ref[0:64, :],
                  preferred_element_type=jnp.float32)
          + jnp.dot(a1hi_ref[...], w1a_ref[64:128, :],
                    preferred_element_type=jnp.float32))
    h1 = jnp.maximum(u1 + b1a_ref[...], 0.0)
    t1 = jnp.dot(h1, w1b_ref[...], preferred_element_type=jnp.float32) + b1b_ref[...]
    u3 = (jnp.dot(a3lo_ref[...], w3a_ref[0:64, :],
                  preferred_element_type=jnp.float32)
          + jnp.dot(a3hi_ref[...], w3a_ref[64:128, :],
                    preferred_element_type=jnp.float32))
    h3 = jnp.maximum(u3 + b3a_ref[...], 0.0)
    t3 = jnp.dot(h3, w3b_ref[...], preferred_element_type=jnp.float32) + b3b_ref[...]
    x = _leaky(t1 + t3)
    out_ref[...] = jnp.dot(x, wfc_ref[...],
                           preferred_element_type=jnp.float32) + bfc_ref[...]


def _layer1_half(agg1_flat, t1, t3, w1a, b1a, w1b, b1b, w3a, b3a, w3b, b3b,
                 wfc, bfc):
    return pl.pallas_call(
        _mlp_pair_fc_body,
        grid=(NBO,),
        in_specs=[
            pl.BlockSpec((R, 64), lambda i, t=t1: (i + 2 * t * NB, 0)),
            pl.BlockSpec((R, 64), lambda i, t=t1: (i + (2 * t + 1) * NB, 0)),
            pl.BlockSpec((R, 64), lambda i, t=t3: (i + 2 * t * NB, 0)),
            pl.BlockSpec((R, 64), lambda i, t=t3: (i + (2 * t + 1) * NB, 0)),
            pl.BlockSpec((H, H), lambda i: (0, 0)),
            pl.BlockSpec((1, H), lambda i: (0, 0)),
            pl.BlockSpec((H, H), lambda i: (0, 0)),
            pl.BlockSpec((1, H), lambda i: (0, 0)),
            pl.BlockSpec((H, H), lambda i: (0, 0)),
            pl.BlockSpec((1, H), lambda i: (0, 0)),
            pl.BlockSpec((H, H), lambda i: (0, 0)),
            pl.BlockSpec((1, H), lambda i: (0, 0)),
            pl.BlockSpec((H, H), lambda i: (0, 0)),
            pl.BlockSpec((1, H), lambda i: (0, 0)),
        ],
        out_specs=pl.BlockSpec((R, H), lambda i: (i, 0)),
        out_shape=jax.ShapeDtypeStruct((N, H), jnp.float32),
    )(agg1_flat, agg1_flat, agg1_flat, agg1_flat,
      w1a, b1a, w1b, b1b, w3a, b3a, w3b, b3b, wfc, bfc)


def _agg0_jnp(x_op_p, x_m7_p, eis):
    out = jnp.zeros((4 * NPAD, 16), jnp.float32)
    srcs = (x_op_p, x_m7_p, x_op_p, x_m7_p)
    for t, (ei, xs) in enumerate(zip(eis, srcs)):
        out = out.at[ei[1] + t * NPAD].add(xs[ei[0]])
    return out


def _agg1_jnp(x_op1, x_m1, eis):
    out = jnp.zeros((8 * NPAD, 64), jnp.float32)
    srcs = (x_op1, x_m1, x_op1, x_m1)
    for t, (ei, xs) in enumerate(zip(eis, srcs)):
        msg = xs[ei[0]]
        out = out.at[ei[1] + 2 * t * NPAD].add(msg[:, :64])
        out = out.at[ei[1] + (2 * t + 1) * NPAD].add(msg[:, 64:])
    return out


def kernel(x_op, x_m, ei_op_op, ei_op_m, ei_m_op, ei_m_m, W_mtrans, b_mtrans,
           W_0_nn1_0, b_0_nn1_0, W_0_nn1_1, b_0_nn1_1,
           W_0_nn2_0, b_0_nn2_0, W_0_nn2_1, b_0_nn2_1,
           W_0_nn3_0, b_0_nn3_0, W_0_nn3_1, b_0_nn3_1,
           W_0_nn4_0, b_0_nn4_0, W_0_nn4_1, b_0_nn4_1,
           W_1_nn1_0, b_1_nn1_0, W_1_nn1_1, b_1_nn1_1,
           W_1_nn2_0, b_1_nn2_0, W_1_nn2_1, b_1_nn2_1,
           W_1_nn3_0, b_1_nn3_0, W_1_nn3_1, b_1_nn3_1,
           W_1_nn4_0, b_1_nn4_0, W_1_nn4_1, b_1_nn4_1,
           W_op_fc, b_op_fc, W_m_fc, b_m_fc):
    f32 = jnp.float32

    x_op_p = jnp.pad(x_op, ((0, NPAD - N), (0, 16 - 7)))
    x_m_pad = jnp.pad(x_m, ((0, NPAD - N), (0, 8 - 4)))
    Wm_p = jnp.pad(W_mtrans, ((0, 8 - 4), (0, 16 - 7)))
    bm_p = jnp.pad(b_mtrans, (0, 16 - 7)).reshape(1, 16)

    def pad_w0(w):
        return jnp.pad(w, ((0, 9), (0, 0)))

    def row(b):
        return b.reshape(1, H)

    eis = (ei_op_op, ei_m_op, ei_op_m, ei_m_m)

    x_m7_p = _prep_mtrans(x_m_pad, Wm_p, bm_p)

    agg0 = _agg0_jnp(x_op_p, x_m7_p, eis)
    x_op1 = _layer0_half(agg0, 0, 1,
                         pad_w0(W_0_nn1_0), row(b_0_nn1_0), W_0_nn1_1, row(b_0_nn1_1),
                         pad_w0(W_0_nn3_0), row(b_0_nn3_0), W_0_nn3_1, row(b_0_nn3_1))
    x_m1 = _layer0_half(agg0, 2, 3,
                        pad_w0(W_0_nn2_0), row(b_0_nn2_0), W_0_nn2_1, row(b_0_nn2_1),
                        pad_w0(W_0_nn4_0), row(b_0_nn4_0), W_0_nn4_1, row(b_0_nn4_1))

    agg1 = _agg1_jnp(x_op1, x_m1, eis)
    out_op = _layer1_half(agg1, 0, 1,
                          W_1_nn1_0, row(b_1_nn1_0), W_1_nn1_1, row(b_1_nn1_1),
                          W_1_nn3_0, row(b_1_nn3_0), W_1_nn3_1, row(b_1_nn3_1),
                          W_op_fc, row(b_op_fc))
    out_m = _layer1_half(agg1, 2, 3,
                         W_1_nn2_0, row(b_1_nn2_0), W_1_nn2_1, row(b_1_nn2_1),
                         W_1_nn4_0, row(b_1_nn4_0), W_1_nn4_1, row(b_1_nn4_1),
                         W_m_fc, row(b_m_fc))
    return out_op.astype(f32), out_m.astype(f32)
